# baseline (speedup 1.0000x reference)
import functools

import jax
import jax.numpy as jnp
from jax import lax
from jax.experimental import pallas as pl
from jax.experimental.pallas import tpu as pltpu
from jax.experimental.pallas import tpu_sc as plsc

_NW = 32
_PW = 3200
_RC = 50
_NQ = _PW // _RC
_NB = 3
_D = 512
_L = 16


def _make_sc_kernel(n):
    mesh = plsc.VectorSubcoreMesh(core_axis_name="c", subcore_axis_name="s")

    @functools.partial(
        pl.kernel,
        mesh=mesh,
        compiler_params=pltpu.CompilerParams(needs_layout_passes=False),
        out_type=jax.ShapeDtypeStruct((n, 1, _D), jnp.float32),
        scratch_types=[
            pltpu.VMEM((_PW,), jnp.int32),
            pltpu.VMEM((_PW,), jnp.float32),
            pltpu.VMEM((128,), jnp.float32),
            pltpu.VMEM((_D,), jnp.float32),
            pltpu.VMEM((_PW,), jnp.float32),
            pltpu.VMEM((_RC, 1, _D), jnp.float32),
            pltpu.VMEM((_RC, 1, _D), jnp.float32),
            pltpu.VMEM((_RC, 1, _D), jnp.float32),
            pltpu.SemaphoreType.DMA,
            pltpu.SemaphoreType.DMA,
            pltpu.SemaphoreType.DMA,
        ],
    )
    def sc_k(z_hbm, nmm_hbm, gyro_hbm, w_hbm, out_hbm,
             z_v, nmm_v, t_v, w_v, s_v, buf0, buf1, buf2, sem0, sem1, sem2):
        wid = lax.axis_index("s") * 2 + lax.axis_index("c")
        base = wid * _PW
        pltpu.sync_copy(z_hbm.at[pl.ds(base, _PW)], z_v)
        pltpu.sync_copy(nmm_hbm.at[pl.ds(base, _PW)], nmm_v)
        pltpu.sync_copy(gyro_hbm, t_v)
        pltpu.sync_copy(w_hbm, w_v)

        @plsc.parallel_loop(0, _PW // _L)
        def s_body(i):
            sl = pl.ds(i * _L, _L)
            g = plsc.load_gather(t_v, [z_v[sl]])
            s_v[sl] = g * nmm_v[sl]

        wl = [w_v[pl.ds(l * _L, _L)] for l in range(_D // _L)]

        def chunk(q, buf, sem):
            row0 = base + q * _RC

            @pl.when((q < _NQ) & (row0 < n))
            def _():
                @pl.when(q >= _NB)
                def _():
                    pltpu.make_async_copy(
                        out_hbm.at[pl.ds(0, _RC)], buf, sem
                    ).wait()

                @plsc.parallel_loop(0, _RC, unroll=2)
                def row_body(j):
                    a = q * _RC + j
                    sj = plsc.load_gather(s_v, [jnp.full((_L,), 0, jnp.int32) + a])
                    for l in range(_D // _L):
                        buf[j, 0, pl.ds(l * _L, _L)] = sj * wl[l]
                pltpu.async_copy(buf, out_hbm.at[pl.ds(row0, _RC)], sem)

        bufs = (buf0, buf1, buf2)
        sems = (sem0, sem1, sem2)

        def q_body(k, carry):
            for b in range(_NB):
                chunk(k * _NB + b, bufs[b], sems[b])
            return carry

        lax.fori_loop(0, (_NQ + _NB - 1) // _NB, q_body, 0)
        for b in range(_NB):
            pltpu.make_async_copy(out_hbm.at[pl.ds(0, _RC)], bufs[b], sems[b]).wait()

    return sc_k


def kernel(Z, nuclear_magnetic_moments, gyro_table, W):
    n = Z.shape[0]
    npad = _NW * _PW
    z_p = jnp.zeros((npad,), jnp.int32).at[:n].set(Z.astype(jnp.int32))
    nmm_p = jnp.zeros((npad,), jnp.float32).at[:n].set(
        nuclear_magnetic_moments[:, 0]
    )
    gyro_pad = jnp.zeros((128,), jnp.float32).at[: gyro_table.shape[0]].set(
        gyro_table[:, 0]
    )
    w_flat = W[:, 0]
    return _make_sc_kernel(n)(z_p, nmm_p, gyro_pad, w_flat)


# speedup vs baseline: 1.0209x; 1.0209x over previous
import functools

import jax
import jax.numpy as jnp
from jax import lax
from jax.experimental import pallas as pl
from jax.experimental.pallas import tpu as pltpu
from jax.experimental.pallas import tpu_sc as plsc

_NW = 32
_AW = 3125
_FW = 3144
_RC = 25
_NQ = _AW // _RC
_D = 512
_L = 16


def _make_sc_kernel(n):
    mesh = plsc.VectorSubcoreMesh(core_axis_name="c", subcore_axis_name="s")

    @functools.partial(
        pl.kernel,
        mesh=mesh,
        compiler_params=pltpu.CompilerParams(needs_layout_passes=False),
        out_type=jax.ShapeDtypeStruct((n, 1, _D), jnp.float32),
        scratch_types=[
            pltpu.VMEM((_FW,), jnp.int32),
            pltpu.VMEM((_FW,), jnp.float32),
            pltpu.VMEM((128,), jnp.float32),
            pltpu.VMEM((_D,), jnp.float32),
            pltpu.VMEM((_FW,), jnp.float32),
            pltpu.VMEM((_RC, 1, _D), jnp.float32),
            pltpu.VMEM((_RC, 1, _D), jnp.float32),
            pltpu.SemaphoreType.DMA,
            pltpu.SemaphoreType.DMA,
        ],
    )
    def sc_k(z_hbm, nmm_hbm, gyro_hbm, w_hbm, out_hbm,
             z_v, nmm_v, t_v, w_v, s_v, buf0, buf1, sem0, sem1):
        wid = lax.axis_index("s") * 2 + lax.axis_index("c")
        base = wid * _AW
        a0 = (base // 8) * 8
        r = base - a0
        pltpu.sync_copy(z_hbm.at[pl.ds(a0, _FW)], z_v)
        pltpu.sync_copy(nmm_hbm.at[pl.ds(a0, _FW)], nmm_v)
        pltpu.sync_copy(gyro_hbm, t_v)
        pltpu.sync_copy(w_hbm, w_v)

        @plsc.parallel_loop(0, _FW // _L)
        def s_body(i):
            src = pl.ds(r + i * _L, _L)
            g = plsc.load_gather(t_v, [z_v[src]])
            s_v[pl.ds(i * _L, _L)] = g * nmm_v[src]

        wl = [w_v[pl.ds(l * _L, _L)] for l in range(_D // _L)]

        def chunk(q, buf, sem):
            @pl.when(q < _NQ)
            def _():
                @pl.when(q >= 2)
                def _():
                    pltpu.make_async_copy(
                        out_hbm.at[pl.ds(0, _RC)], buf, sem
                    ).wait()

                @plsc.parallel_loop(0, _RC, unroll=2)
                def row_body(j):
                    a = q * _RC + j
                    sj = plsc.load_gather(s_v, [jnp.full((_L,), 0, jnp.int32) + a])
                    for l in range(_D // _L):
                        buf[j, 0, pl.ds(l * _L, _L)] = sj * wl[l]

                pltpu.async_copy(buf, out_hbm.at[pl.ds(base + q * _RC, _RC)], sem)

        def q_body(k, carry):
            chunk(k * 2, buf0, sem0)
            chunk(k * 2 + 1, buf1, sem1)
            return carry

        lax.fori_loop(0, (_NQ + 1) // 2, q_body, 0)
        pltpu.make_async_copy(out_hbm.at[pl.ds(0, _RC)], buf0, sem0).wait()
        pltpu.make_async_copy(out_hbm.at[pl.ds(0, _RC)], buf1, sem1).wait()

    return sc_k


def kernel(Z, nuclear_magnetic_moments, gyro_table, W):
    n = Z.shape[0]
    npad = _NW * _AW + _FW
    z_p = jnp.zeros((npad,), jnp.int32).at[:n].set(Z.astype(jnp.int32))
    nmm_p = jnp.zeros((npad,), jnp.float32).at[:n].set(
        nuclear_magnetic_moments[:, 0]
    )
    gyro_pad = jnp.zeros((128,), jnp.float32).at[: gyro_table.shape[0]].set(
        gyro_table[:, 0]
    )
    w_flat = W[:, 0]
    return _make_sc_kernel(n)(z_p, nmm_p, gyro_pad, w_flat)


# speedup vs baseline: 1.0223x; 1.0014x over previous
import functools

import jax
import jax.numpy as jnp
from jax import lax
from jax.experimental import pallas as pl
from jax.experimental.pallas import tpu as pltpu
from jax.experimental.pallas import tpu_sc as plsc

_NW = 32
_PW = 3200
_RC = 80
_NQ = _PW // _RC
_D = 512
_L = 16


def _make_sc_kernel(n):
    mesh = plsc.VectorSubcoreMesh(core_axis_name="c", subcore_axis_name="s")

    @functools.partial(
        pl.kernel,
        mesh=mesh,
        compiler_params=pltpu.CompilerParams(needs_layout_passes=False),
        out_type=jax.ShapeDtypeStruct((n, 1, _D), jnp.float32),
        scratch_types=[
            pltpu.VMEM((_PW,), jnp.int32),
            pltpu.VMEM((_PW,), jnp.float32),
            pltpu.VMEM((128,), jnp.float32),
            pltpu.VMEM((_D,), jnp.float32),
            pltpu.VMEM((_PW,), jnp.float32),
            pltpu.VMEM((_RC, 1, _D), jnp.float32),
            pltpu.VMEM((_RC, 1, _D), jnp.float32),
            pltpu.SemaphoreType.DMA,
            pltpu.SemaphoreType.DMA,
        ],
    )
    def sc_k(z_hbm, nmm_hbm, gyro_hbm, w_hbm, out_hbm,
             z_v, nmm_v, t_v, w_v, s_v, buf0, buf1, sem0, sem1):
        wid = lax.axis_index("s") * 2 + lax.axis_index("c")
        base = wid * _PW
        pltpu.sync_copy(z_hbm.at[pl.ds(base, _PW)], z_v)
        pltpu.sync_copy(nmm_hbm.at[pl.ds(base, _PW)], nmm_v)
        pltpu.sync_copy(gyro_hbm, t_v)
        pltpu.sync_copy(w_hbm, w_v)

        @plsc.parallel_loop(0, _PW // _L)
        def s_body(i):
            sl = pl.ds(i * _L, _L)
            g = plsc.load_gather(t_v, [z_v[sl]])
            s_v[sl] = g * nmm_v[sl]

        wl = [w_v[pl.ds(l * _L, _L)] for l in range(_D // _L)]

        def chunk(q, buf, sem):
            row0 = base + q * _RC

            @pl.when(row0 < n)
            def _():
                @pl.when(q >= 2)
                def _():
                    pltpu.make_async_copy(
                        out_hbm.at[pl.ds(0, _RC)], buf, sem
                    ).wait()

                @plsc.parallel_loop(0, _RC, unroll=2)
                def row_body(j):
                    a = q * _RC + j
                    sj = plsc.load_gather(s_v, [jnp.full((_L,), 0, jnp.int32) + a])
                    for l in range(_D // _L):
                        buf[j, 0, pl.ds(l * _L, _L)] = sj * wl[l]
                pltpu.async_copy(buf, out_hbm.at[pl.ds(row0, _RC)], sem)

        def q_body(k, carry):
            chunk(k * 2, buf0, sem0)
            chunk(k * 2 + 1, buf1, sem1)
            return carry

        lax.fori_loop(0, _NQ // 2, q_body, 0)
        pltpu.make_async_copy(out_hbm.at[pl.ds(0, _RC)], buf0, sem0).wait()
        pltpu.make_async_copy(out_hbm.at[pl.ds(0, _RC)], buf1, sem1).wait()

    return sc_k


def kernel(Z, nuclear_magnetic_moments, gyro_table, W):
    n = Z.shape[0]
    npad = _NW * _PW
    z_p = jnp.zeros((npad,), jnp.int32).at[:n].set(Z.astype(jnp.int32))
    nmm_p = jnp.zeros((npad,), jnp.float32).at[:n].set(
        nuclear_magnetic_moments[:, 0]
    )
    gyro_pad = jnp.zeros((128,), jnp.float32).at[: gyro_table.shape[0]].set(
        gyro_table[:, 0]
    )
    w_flat = W[:, 0]
    return _make_sc_kernel(n)(z_p, nmm_p, gyro_pad, w_flat)
